# baseline (speedup 1.0000x reference)
import functools

import jax
import jax.numpy as jnp
from jax import lax
from jax.experimental import pallas as pl
from jax.experimental.pallas import tpu as pltpu
from jax.experimental.pallas import tpu_sc as plsc

COMMIT = 0.25

BI = 512
BJ = 8192

NC = 2
NS = 16
NW = NC * NS


def _dist_body(nj, x_ref, embt_ref, w_ref, b_ref, dist_ref, idx_ref, cb_ref,
               cbt_s, cn_s, min_s, arg_s):
    j = pl.program_id(0)
    i = pl.program_id(1)
    bi = x_ref.shape[0]
    bj = embt_ref.shape[1]

    @pl.when(i == 0)
    def _():
        cbt = jnp.dot(w_ref[...], embt_ref[...]) + b_ref[...]
        cbt_s[...] = cbt
        cn_s[...] = jnp.sum(cbt * cbt, axis=0, keepdims=True)
        cb_ref[...] = jnp.transpose(cbt, (1, 0))

    x = x_ref[...]
    xn = jnp.sum(x * x, axis=1, keepdims=True)
    prod = jnp.dot(x, cbt_s[...])
    dist = xn + cn_s[...] - 2.0 * prod
    dist_ref[...] = dist

    bmin = jnp.min(dist, axis=1, keepdims=True)
    cols = lax.broadcasted_iota(jnp.int32, dist.shape, 1)
    barg = jnp.min(jnp.where(dist == bmin, cols, jnp.int32(2**31 - 1)),
                   axis=1, keepdims=True) + j * bj
    row = pl.ds(i * bi, bi)

    @pl.when(j == 0)
    def _():
        min_s[row] = bmin
        arg_s[row] = barg

    @pl.when(j > 0)
    def _():
        better = bmin < min_s[row]
        min_s[row] = jnp.where(better, bmin, min_s[row])
        arg_s[row] = jnp.where(better, barg, arg_s[row])

    @pl.when(j == nj - 1)
    def _():
        idx_ref[...] = arg_s[row]


def _distances(flat, embt, w, b_col):
    n_pts, d = flat.shape
    n_emb = embt.shape[1]
    ni = n_pts // BI
    nj = n_emb // BJ
    return pl.pallas_call(
        functools.partial(_dist_body, nj),
        grid=(nj, ni),
        in_specs=[
            pl.BlockSpec((BI, d), lambda j, i: (i, 0)),
            pl.BlockSpec((d, BJ), lambda j, i: (0, j)),
            pl.BlockSpec((d, d), lambda j, i: (0, 0)),
            pl.BlockSpec((d, 1), lambda j, i: (0, 0)),
        ],
        out_specs=[
            pl.BlockSpec((BI, BJ), lambda j, i: (i, j)),
            pl.BlockSpec((BI, 1), lambda j, i: (i, 0)),
            pl.BlockSpec((BJ, d), lambda j, i: (j, 0)),
        ],
        out_shape=[
            jax.ShapeDtypeStruct((n_pts, n_emb), jnp.float32),
            jax.ShapeDtypeStruct((n_pts, 1), jnp.int32),
            jax.ShapeDtypeStruct((n_emb, d), jnp.float32),
        ],
        scratch_shapes=[
            pltpu.VMEM((d, BJ), jnp.float32),
            pltpu.VMEM((1, BJ), jnp.float32),
            pltpu.VMEM((n_pts, 1), jnp.float32),
            pltpu.VMEM((n_pts, 1), jnp.int32),
        ],
    )(flat, embt, w, b_col)


def _sc_body(n_emb, per_w, cb_hbm, idx_hbm, zq_hbm, hist_hbm,
             idx_v, rows_v, rowst_v, hist_v, sem):
    d = cb_hbm.shape[1]
    wid = lax.axis_index("s") * NC + lax.axis_index("c")
    base = wid * per_w
    per_b = zq_hbm.shape[2]
    b = base // per_b
    hw0 = base % per_b
    pltpu.sync_copy(idx_hbm.at[pl.ds(base, per_w)], idx_v)

    chunk = 128
    copies = []
    for k in range(per_w // chunk):
        sl = pl.ds(k * chunk, chunk)
        copies.append(pltpu.async_copy(cb_hbm.at[idx_v.at[sl]],
                                       rows_v.at[sl], sem))
    for cp in copies:
        cp.wait()

    def tbody(dd, carry):
        col = jnp.full((16,), dd, jnp.int32)
        for t in range(per_w // 16):
            row = t * 16 + lax.iota(jnp.int32, 16)
            v = plsc.load_gather(rows_v, [row, col])
            rowst_v[dd, pl.ds(t * 16, 16)] = v
        return carry

    lax.fori_loop(0, d, tbody, 0)
    pltpu.sync_copy(rowst_v, zq_hbm.at[b, :, pl.ds(hw0, per_w)])

    zeros = jnp.zeros((16,), jnp.float32)

    def zbody(k, carry):
        for t in range(16):
            hist_v[pl.ds(k * 256 + t * 16, 16)] = zeros
        return carry

    lax.fori_loop(0, n_emb // 256, zbody, 0)
    ones = jnp.ones((16,), jnp.float32)

    def hbody(k, carry):
        iv = idx_v[pl.ds(k * 16, 16)]
        plsc.addupdate_scatter(hist_v, [iv], ones)
        return carry

    lax.fori_loop(0, per_w // 16, hbody, 0)
    pltpu.sync_copy(hist_v, hist_hbm.at[wid])


def _sc_gather_hist(cb, idx, nb, d, per_b):
    n_emb = cb.shape[0]
    n_pts = idx.shape[0]
    per_w = n_pts // NW
    mesh = plsc.VectorSubcoreMesh(core_axis_name="c", subcore_axis_name="s",
                                  num_cores=NC, num_subcores=NS)
    fn = pl.kernel(
        functools.partial(_sc_body, n_emb, per_w),
        mesh=mesh,
        out_type=[
            jax.ShapeDtypeStruct((nb, d, per_b), jnp.float32),
            jax.ShapeDtypeStruct((NW, n_emb), jnp.float32),
        ],
        scratch_types=[
            pltpu.VMEM((per_w,), jnp.int32),
            pltpu.VMEM((per_w, d), jnp.float32),
            pltpu.VMEM((d, per_w), jnp.float32),
            pltpu.VMEM((n_emb,), jnp.float32),
            pltpu.SemaphoreType.DMA,
        ],
        compiler_params=pltpu.CompilerParams(needs_layout_passes=False,
                                             use_tc_tiling_on_sc=False),
    )
    return fn(cb, idx)


def _finalize_body(ze_ref, zq_ref, hist_ref, loss_ref, perp_ref):
    df = zq_ref[...] - ze_ref[...]
    n = df.shape[0] * df.shape[1] * df.shape[2]
    n_pts = df.shape[0] * df.shape[2]
    loss_ref[...] = ((1.0 + COMMIT) * (jnp.sum(df * df) / n)).reshape(1, 1)
    counts = jnp.sum(hist_ref[...], axis=0, keepdims=True)
    p = counts / n_pts
    ent = jnp.sum(p * jnp.log(p + 1e-10))
    perp_ref[...] = jnp.exp(-ent).reshape(1, 1)


def _finalize(ze4, zq, hist):
    return pl.pallas_call(
        _finalize_body,
        out_shape=[
            jax.ShapeDtypeStruct((1, 1), jnp.float32),
            jax.ShapeDtypeStruct((1, 1), jnp.float32),
        ],
    )(ze4, zq, hist)


def kernel(z_e, embedding, W_proj, b_proj):
    B, D, H, W = z_e.shape
    flat = jnp.transpose(z_e, (0, 2, 3, 1)).reshape(-1, D)
    embt = embedding.T
    b_col = b_proj.reshape(D, 1)

    dist, idx2, cb = _distances(flat, embt, W_proj, b_col)
    idx = idx2.reshape(-1)

    zq4, hist = _sc_gather_hist(cb, idx, B, D, H * W)
    ze4 = z_e.reshape(B, D, H * W)
    loss2, perp2 = _finalize(ze4, zq4, hist)

    z_q_out = zq4.reshape(B, D, H, W)
    return (z_q_out, loss2.reshape(()), perp2.reshape(()), idx, dist)


# speedup vs baseline: 1.0030x; 1.0030x over previous
import functools

import jax
import jax.numpy as jnp
from jax import lax
from jax.experimental import pallas as pl
from jax.experimental.pallas import tpu as pltpu
from jax.experimental.pallas import tpu_sc as plsc

COMMIT = 0.25

BI = 512
BJ = 8192

NC = 2
NS = 16
NW = NC * NS


def _dist_body(nj, x_ref, embt_ref, w_ref, b_ref, dist_ref, idx_ref, cb_ref,
               minv_ref, cbt_s, cn_s, min_s, arg_s):
    j = pl.program_id(0)
    i = pl.program_id(1)
    bi = x_ref.shape[0]
    bj = embt_ref.shape[1]

    @pl.when(i == 0)
    def _():
        cbt = jnp.dot(w_ref[...], embt_ref[...]) + b_ref[...]
        cbt_s[...] = cbt
        cn_s[...] = jnp.sum(cbt * cbt, axis=0, keepdims=True)
        cb_ref[...] = jnp.transpose(cbt, (1, 0))

    x = x_ref[...]
    xn = jnp.sum(x * x, axis=1, keepdims=True)
    prod = jnp.dot(x, cbt_s[...])
    dist = xn + cn_s[...] - 2.0 * prod
    dist_ref[...] = dist

    bmin = jnp.min(dist, axis=1, keepdims=True)
    cols = lax.broadcasted_iota(jnp.int32, dist.shape, 1)
    barg = jnp.min(jnp.where(dist == bmin, cols, jnp.int32(2**31 - 1)),
                   axis=1, keepdims=True) + j * bj
    row = pl.ds(i * bi, bi)

    @pl.when(j == 0)
    def _():
        min_s[row] = bmin
        arg_s[row] = barg

    @pl.when(j > 0)
    def _():
        better = bmin < min_s[row]
        min_s[row] = jnp.where(better, bmin, min_s[row])
        arg_s[row] = jnp.where(better, barg, arg_s[row])

    @pl.when(j == nj - 1)
    def _():
        idx_ref[...] = arg_s[row]
        minv_ref[...] = min_s[row]


def _distances(flat, embt, w, b_col):
    n_pts, d = flat.shape
    n_emb = embt.shape[1]
    ni = n_pts // BI
    nj = n_emb // BJ
    return pl.pallas_call(
        functools.partial(_dist_body, nj),
        grid=(nj, ni),
        in_specs=[
            pl.BlockSpec((BI, d), lambda j, i: (i, 0)),
            pl.BlockSpec((d, BJ), lambda j, i: (0, j)),
            pl.BlockSpec((d, d), lambda j, i: (0, 0)),
            pl.BlockSpec((d, 1), lambda j, i: (0, 0)),
        ],
        out_specs=[
            pl.BlockSpec((BI, BJ), lambda j, i: (i, j)),
            pl.BlockSpec((BI, 1), lambda j, i: (i, 0)),
            pl.BlockSpec((BJ, d), lambda j, i: (j, 0)),
            pl.BlockSpec((BI, 1), lambda j, i: (i, 0)),
        ],
        out_shape=[
            jax.ShapeDtypeStruct((n_pts, n_emb), jnp.float32),
            jax.ShapeDtypeStruct((n_pts, 1), jnp.int32),
            jax.ShapeDtypeStruct((n_emb, d), jnp.float32),
            jax.ShapeDtypeStruct((n_pts, 1), jnp.float32),
        ],
        scratch_shapes=[
            pltpu.VMEM((d, BJ), jnp.float32),
            pltpu.VMEM((1, BJ), jnp.float32),
            pltpu.VMEM((n_pts, 1), jnp.float32),
            pltpu.VMEM((n_pts, 1), jnp.int32),
        ],
    )(flat, embt, w, b_col)


def _sc_body(n_emb, per_w, cb_hbm, idx_hbm, zq_hbm, hist_hbm,
             idx_v, rows_v, rowst_v, hist_v, sem):
    d = cb_hbm.shape[1]
    wid = lax.axis_index("s") * NC + lax.axis_index("c")
    base = wid * per_w
    per_b = zq_hbm.shape[2]
    b = base // per_b
    hw0 = base % per_b
    pltpu.sync_copy(idx_hbm.at[pl.ds(base, per_w)], idx_v)

    chunk = 128
    copies = []
    for k in range(per_w // chunk):
        sl = pl.ds(k * chunk, chunk)
        copies.append(pltpu.async_copy(cb_hbm.at[idx_v.at[sl]],
                                       rows_v.at[sl], sem))
    for cp in copies:
        cp.wait()

    def tbody(dd, carry):
        col = jnp.full((16,), dd, jnp.int32)
        for t in range(per_w // 16):
            row = t * 16 + lax.iota(jnp.int32, 16)
            v = plsc.load_gather(rows_v, [row, col])
            rowst_v[dd, pl.ds(t * 16, 16)] = v
        return carry

    lax.fori_loop(0, d, tbody, 0)
    pltpu.sync_copy(rowst_v, zq_hbm.at[b, :, pl.ds(hw0, per_w)])

    zeros = jnp.zeros((16,), jnp.float32)

    def zbody(k, carry):
        for t in range(16):
            hist_v[pl.ds(k * 256 + t * 16, 16)] = zeros
        return carry

    lax.fori_loop(0, n_emb // 256, zbody, 0)
    ones = jnp.ones((16,), jnp.float32)

    def hbody(k, carry):
        iv = idx_v[pl.ds(k * 16, 16)]
        plsc.addupdate_scatter(hist_v, [iv], ones)
        return carry

    lax.fori_loop(0, per_w // 16, hbody, 0)
    pltpu.sync_copy(hist_v, hist_hbm.at[wid])


def _sc_gather_hist(cb, idx, nb, d, per_b):
    n_emb = cb.shape[0]
    n_pts = idx.shape[0]
    per_w = n_pts // NW
    mesh = plsc.VectorSubcoreMesh(core_axis_name="c", subcore_axis_name="s",
                                  num_cores=NC, num_subcores=NS)
    fn = pl.kernel(
        functools.partial(_sc_body, n_emb, per_w),
        mesh=mesh,
        out_type=[
            jax.ShapeDtypeStruct((nb, d, per_b), jnp.float32),
            jax.ShapeDtypeStruct((NW, n_emb), jnp.float32),
        ],
        scratch_types=[
            pltpu.VMEM((per_w,), jnp.int32),
            pltpu.VMEM((per_w, d), jnp.float32),
            pltpu.VMEM((d, per_w), jnp.float32),
            pltpu.VMEM((n_emb,), jnp.float32),
            pltpu.SemaphoreType.DMA,
        ],
        compiler_params=pltpu.CompilerParams(needs_layout_passes=False,
                                             use_tc_tiling_on_sc=False),
    )
    return fn(cb, idx)


def _finalize_body(d, minv_ref, hist_ref, loss_ref, perp_ref):
    n_pts = minv_ref.shape[0]
    loss_ref[...] = ((1.0 + COMMIT)
                     * (jnp.sum(minv_ref[...]) / (n_pts * d))).reshape(1, 1)
    counts = jnp.sum(hist_ref[...], axis=0, keepdims=True)
    p = counts / n_pts
    ent = jnp.sum(p * jnp.log(p + 1e-10))
    perp_ref[...] = jnp.exp(-ent).reshape(1, 1)


def _finalize(minv, hist, d):
    return pl.pallas_call(
        functools.partial(_finalize_body, d),
        out_shape=[
            jax.ShapeDtypeStruct((1, 1), jnp.float32),
            jax.ShapeDtypeStruct((1, 1), jnp.float32),
        ],
    )(minv, hist)


def kernel(z_e, embedding, W_proj, b_proj):
    B, D, H, W = z_e.shape
    flat = jnp.transpose(z_e, (0, 2, 3, 1)).reshape(-1, D)
    embt = embedding.T
    b_col = b_proj.reshape(D, 1)

    dist, idx2, cb, minv = _distances(flat, embt, W_proj, b_col)
    idx = idx2.reshape(-1)

    zq4, hist = _sc_gather_hist(cb, idx, B, D, H * W)
    loss2, perp2 = _finalize(minv, hist, D)

    z_q_out = zq4.reshape(B, D, H, W)
    return (z_q_out, loss2.reshape(()), perp2.reshape(()), idx, dist)


# speedup vs baseline: 1.0265x; 1.0235x over previous
import functools

import jax
import jax.numpy as jnp
from jax import lax
from jax.experimental import pallas as pl
from jax.experimental.pallas import tpu as pltpu
from jax.experimental.pallas import tpu_sc as plsc

COMMIT = 0.25

BI = 512
BJ = 8192

NC = 2
NS = 16
NW = NC * NS


def _dist_body(nj, x_ref, embt_ref, w_ref, b_ref, dist_ref, idx_ref, cb_ref,
               minv_ref, cbt_s, cn_s, min_s, arg_s):
    j = pl.program_id(0)
    i = pl.program_id(1)
    bi = x_ref.shape[0]
    bj = embt_ref.shape[1]

    @pl.when(i == 0)
    def _():
        cbt = jnp.dot(w_ref[...], embt_ref[...]) + b_ref[...]
        cbt_s[...] = cbt
        cn_s[...] = jnp.sum(cbt * cbt, axis=0, keepdims=True)
        cb_ref[...] = jnp.transpose(cbt, (1, 0))

    x = x_ref[...]
    xn = jnp.sum(x * x, axis=1, keepdims=True)
    prod = jnp.dot(x * -2.0, cbt_s[...])
    dist = xn + cn_s[...] + prod
    dist_ref[...] = dist

    bmin = jnp.min(dist, axis=1, keepdims=True)
    cols = lax.broadcasted_iota(jnp.int32, dist.shape, 1)
    barg = jnp.min(jnp.where(dist == bmin, cols, jnp.int32(2**31 - 1)),
                   axis=1, keepdims=True) + j * bj
    row = pl.ds(i * bi, bi)

    @pl.when(j == 0)
    def _():
        min_s[row] = bmin
        arg_s[row] = barg

    @pl.when(j > 0)
    def _():
        better = bmin < min_s[row]
        min_s[row] = jnp.where(better, bmin, min_s[row])
        arg_s[row] = jnp.where(better, barg, arg_s[row])

    @pl.when(j == nj - 1)
    def _():
        idx_ref[...] = jnp.reshape(arg_s[row], idx_ref.shape)
        minv_ref[...] = jnp.reshape(min_s[row], minv_ref.shape)


def _distances(flat, embt, w, b_col):
    n_pts, d = flat.shape
    n_emb = embt.shape[1]
    ni = n_pts // BI
    nj = n_emb // BJ
    return pl.pallas_call(
        functools.partial(_dist_body, nj),
        grid=(nj, ni),
        in_specs=[
            pl.BlockSpec((BI, d), lambda j, i: (i, 0)),
            pl.BlockSpec((d, BJ), lambda j, i: (0, j)),
            pl.BlockSpec((d, d), lambda j, i: (0, 0)),
            pl.BlockSpec((d, 1), lambda j, i: (0, 0)),
        ],
        out_specs=[
            pl.BlockSpec((BI, BJ), lambda j, i: (i, j)),
            pl.BlockSpec((1, BI // 128, 128), lambda j, i: (i, 0, 0)),
            pl.BlockSpec((BJ, d), lambda j, i: (j, 0)),
            pl.BlockSpec((1, BI // 128, 128), lambda j, i: (i, 0, 0)),
        ],
        out_shape=[
            jax.ShapeDtypeStruct((n_pts, n_emb), jnp.float32),
            jax.ShapeDtypeStruct((ni, BI // 128, 128), jnp.int32),
            jax.ShapeDtypeStruct((n_emb, d), jnp.float32),
            jax.ShapeDtypeStruct((ni, BI // 128, 128), jnp.float32),
        ],
        scratch_shapes=[
            pltpu.VMEM((d, BJ), jnp.float32),
            pltpu.VMEM((1, BJ), jnp.float32),
            pltpu.VMEM((n_pts, 1), jnp.float32),
            pltpu.VMEM((n_pts, 1), jnp.int32),
        ],
    )(flat, embt, w, b_col)


def _sc_body(n_emb, per_w, cb_hbm, idx_hbm, zq_hbm, hist_hbm,
             idx_v, rows_v, hist_v, sem):
    wid = lax.axis_index("s") * NC + lax.axis_index("c")
    base = wid * per_w
    pltpu.sync_copy(idx_hbm.at[pl.ds(base, per_w)], idx_v)

    chunk = 128
    copies = []
    for k in range(per_w // chunk):
        sl = pl.ds(k * chunk, chunk)
        copies.append(pltpu.async_copy(cb_hbm.at[idx_v.at[sl]],
                                       rows_v.at[sl], sem))
    for cp in copies:
        cp.wait()
    pltpu.sync_copy(rows_v, zq_hbm.at[pl.ds(base, per_w)])

    zeros = jnp.zeros((16,), jnp.float32)

    def zbody(k, carry):
        for t in range(16):
            hist_v[pl.ds(k * 256 + t * 16, 16)] = zeros
        return carry

    lax.fori_loop(0, n_emb // 256, zbody, 0)
    ones = jnp.ones((16,), jnp.float32)

    def hbody(k, carry):
        iv = idx_v[pl.ds(k * 16, 16)]
        plsc.addupdate_scatter(hist_v, [iv], ones)
        return carry

    lax.fori_loop(0, per_w // 16, hbody, 0)
    pltpu.sync_copy(hist_v, hist_hbm.at[wid])


def _sc_gather_hist(cb, idx):
    n_emb, d = cb.shape
    n_pts = idx.shape[0]
    per_w = n_pts // NW
    mesh = plsc.VectorSubcoreMesh(core_axis_name="c", subcore_axis_name="s",
                                  num_cores=NC, num_subcores=NS)
    fn = pl.kernel(
        functools.partial(_sc_body, n_emb, per_w),
        mesh=mesh,
        out_type=[
            jax.ShapeDtypeStruct((n_pts, d), jnp.float32),
            jax.ShapeDtypeStruct((NW, n_emb), jnp.float32),
        ],
        scratch_types=[
            pltpu.VMEM((per_w,), jnp.int32),
            pltpu.VMEM((per_w, d), jnp.float32),
            pltpu.VMEM((n_emb,), jnp.float32),
            pltpu.SemaphoreType.DMA,
        ],
        compiler_params=pltpu.CompilerParams(needs_layout_passes=False,
                                             use_tc_tiling_on_sc=False),
    )
    return fn(cb, idx)


def _finalize_body(d, minv_ref, hist_ref, loss_ref, perp_ref):
    n_pts = 1
    for s in minv_ref.shape:
        n_pts *= s
    loss_ref[...] = ((1.0 + COMMIT)
                     * (jnp.sum(minv_ref[...]) / (n_pts * d))).reshape(1, 1)
    counts = jnp.sum(hist_ref[...], axis=0, keepdims=True)
    p = counts / n_pts
    ent = jnp.sum(p * jnp.log(p + 1e-10))
    perp_ref[...] = jnp.exp(-ent).reshape(1, 1)


def _finalize(minv, hist, d):
    return pl.pallas_call(
        functools.partial(_finalize_body, d),
        out_shape=[
            jax.ShapeDtypeStruct((1, 1), jnp.float32),
            jax.ShapeDtypeStruct((1, 1), jnp.float32),
        ],
    )(minv, hist)


def kernel(z_e, embedding, W_proj, b_proj):
    B, D, H, W = z_e.shape
    flat = jnp.transpose(z_e, (0, 2, 3, 1)).reshape(-1, D)
    embt = embedding.T
    b_col = b_proj.reshape(D, 1)

    dist, idx2, cb, minv = _distances(flat, embt, W_proj, b_col)
    idx = idx2.reshape(-1)

    zq_flat, hist = _sc_gather_hist(cb, idx)
    loss2, perp2 = _finalize(minv.reshape(minv.shape[0] * minv.shape[1], 128),
                             hist, D)

    z_q_out = jnp.transpose(zq_flat.reshape(B, H, W, D), (0, 3, 1, 2))
    return (z_q_out, loss2.reshape(()), perp2.reshape(()), idx, dist)


# speedup vs baseline: 1.0371x; 1.0104x over previous
import functools

import jax
import jax.numpy as jnp
from jax import lax
from jax.experimental import pallas as pl
from jax.experimental.pallas import tpu as pltpu
from jax.experimental.pallas import tpu_sc as plsc

COMMIT = 0.25

BI = 512
BJ = 8192

NC = 2
NS = 16
NW = NC * NS


def _dist_body(nj, ze_ref, embt_ref, w_ref, b_ref, dist_ref, idx_ref, cb_ref,
               minv_ref, cbt_s, cn_s, min_s, arg_s):
    j = pl.program_id(0)
    i = pl.program_id(1)
    d = ze_ref.shape[1]
    bi = ze_ref.shape[2] * ze_ref.shape[3]
    bj = embt_ref.shape[1]

    @pl.when(i == 0)
    def _():
        cbt = jnp.dot(w_ref[...], embt_ref[...]) + b_ref[...]
        cbt_s[...] = cbt
        cn_s[...] = jnp.sum(cbt * cbt, axis=0, keepdims=True)
        cb_ref[...] = jnp.transpose(cbt, (1, 0))

    x = jnp.transpose(ze_ref[...].reshape(d, bi), (1, 0))
    xn = jnp.sum(x * x, axis=1, keepdims=True)
    prod = jnp.dot(x * -2.0, cbt_s[...])
    dist = xn + cn_s[...] + prod
    dist_ref[...] = dist

    bmin = jnp.min(dist, axis=1, keepdims=True)
    cols = lax.broadcasted_iota(jnp.int32, dist.shape, 1)
    barg = jnp.min(jnp.where(dist == bmin, cols, jnp.int32(2**31 - 1)),
                   axis=1, keepdims=True) + j * bj
    row = pl.ds(i * bi, bi)

    @pl.when(j == 0)
    def _():
        min_s[row] = bmin
        arg_s[row] = barg

    @pl.when(j > 0)
    def _():
        better = bmin < min_s[row]
        min_s[row] = jnp.where(better, bmin, min_s[row])
        arg_s[row] = jnp.where(better, barg, arg_s[row])

    @pl.when(j == nj - 1)
    def _():
        idx_ref[...] = jnp.reshape(arg_s[row], idx_ref.shape)
        minv_ref[...] = jnp.reshape(min_s[row], minv_ref.shape)


def _distances(z_e, embt, w, b_col):
    nb, d, h, wdim = z_e.shape
    n_pts = nb * h * wdim
    n_emb = embt.shape[1]
    ni = n_pts // BI
    nj = n_emb // BJ
    blocks_per_b = (h * wdim) // BI
    h_blk = h // blocks_per_b
    return pl.pallas_call(
        functools.partial(_dist_body, nj),
        grid=(nj, ni),
        in_specs=[
            pl.BlockSpec((1, d, h_blk, wdim),
                         lambda j, i: (i // blocks_per_b, 0,
                                       i % blocks_per_b, 0)),
            pl.BlockSpec((d, BJ), lambda j, i: (0, j)),
            pl.BlockSpec((d, d), lambda j, i: (0, 0)),
            pl.BlockSpec((d, 1), lambda j, i: (0, 0)),
        ],
        out_specs=[
            pl.BlockSpec((BI, BJ), lambda j, i: (i, j)),
            pl.BlockSpec((1, BI // 128, 128), lambda j, i: (i, 0, 0)),
            pl.BlockSpec((BJ, d), lambda j, i: (j, 0)),
            pl.BlockSpec((1, BI // 128, 128), lambda j, i: (i, 0, 0)),
        ],
        out_shape=[
            jax.ShapeDtypeStruct((n_pts, n_emb), jnp.float32),
            jax.ShapeDtypeStruct((ni, BI // 128, 128), jnp.int32),
            jax.ShapeDtypeStruct((n_emb, d), jnp.float32),
            jax.ShapeDtypeStruct((ni, BI // 128, 128), jnp.float32),
        ],
        scratch_shapes=[
            pltpu.VMEM((d, BJ), jnp.float32),
            pltpu.VMEM((1, BJ), jnp.float32),
            pltpu.VMEM((n_pts, 1), jnp.float32),
            pltpu.VMEM((n_pts, 1), jnp.int32),
        ],
    )(z_e, embt, w, b_col)


def _sc_body(n_emb, per_w, cb_hbm, idx_hbm, zq_hbm, hist_hbm,
             idx_v, rows_v, hist_v, sem):
    wid = lax.axis_index("s") * NC + lax.axis_index("c")
    base = wid * per_w
    pltpu.sync_copy(idx_hbm.at[pl.ds(base, per_w)], idx_v)

    chunk = 128
    copies = []
    for k in range(per_w // chunk):
        sl = pl.ds(k * chunk, chunk)
        copies.append(pltpu.async_copy(cb_hbm.at[idx_v.at[sl]],
                                       rows_v.at[sl], sem))
    for cp in copies:
        cp.wait()
    pltpu.sync_copy(rows_v, zq_hbm.at[pl.ds(base, per_w)])

    zeros = jnp.zeros((16,), jnp.float32)

    def zbody(k, carry):
        for t in range(16):
            hist_v[pl.ds(k * 256 + t * 16, 16)] = zeros
        return carry

    lax.fori_loop(0, n_emb // 256, zbody, 0)
    ones = jnp.ones((16,), jnp.float32)

    def hbody(k, carry):
        iv = idx_v[pl.ds(k * 16, 16)]
        plsc.addupdate_scatter(hist_v, [iv], ones)
        return carry

    lax.fori_loop(0, per_w // 16, hbody, 0)
    pltpu.sync_copy(hist_v, hist_hbm.at[wid])


def _sc_gather_hist(cb, idx):
    n_emb, d = cb.shape
    n_pts = idx.shape[0]
    per_w = n_pts // NW
    mesh = plsc.VectorSubcoreMesh(core_axis_name="c", subcore_axis_name="s",
                                  num_cores=NC, num_subcores=NS)
    fn = pl.kernel(
        functools.partial(_sc_body, n_emb, per_w),
        mesh=mesh,
        out_type=[
            jax.ShapeDtypeStruct((n_pts, d), jnp.float32),
            jax.ShapeDtypeStruct((NW, n_emb), jnp.float32),
        ],
        scratch_types=[
            pltpu.VMEM((per_w,), jnp.int32),
            pltpu.VMEM((per_w, d), jnp.float32),
            pltpu.VMEM((n_emb,), jnp.float32),
            pltpu.SemaphoreType.DMA,
        ],
        compiler_params=pltpu.CompilerParams(needs_layout_passes=False,
                                             use_tc_tiling_on_sc=False),
    )
    return fn(cb, idx)


def _finalize_body(d, minv_ref, hist_ref, loss_ref, perp_ref):
    n_pts = 1
    for s in minv_ref.shape:
        n_pts *= s
    loss_ref[...] = ((1.0 + COMMIT)
                     * (jnp.sum(minv_ref[...]) / (n_pts * d))).reshape(1, 1)
    counts = jnp.sum(hist_ref[...], axis=0, keepdims=True)
    p = counts / n_pts
    ent = jnp.sum(p * jnp.log(p + 1e-10))
    perp_ref[...] = jnp.exp(-ent).reshape(1, 1)


def _finalize(minv, hist, d):
    return pl.pallas_call(
        functools.partial(_finalize_body, d),
        out_shape=[
            jax.ShapeDtypeStruct((1, 1), jnp.float32),
            jax.ShapeDtypeStruct((1, 1), jnp.float32),
        ],
    )(minv, hist)


def kernel(z_e, embedding, W_proj, b_proj):
    B, D, H, W = z_e.shape
    embt = embedding.T
    b_col = b_proj.reshape(D, 1)

    dist, idx2, cb, minv = _distances(z_e, embt, W_proj, b_col)
    idx = idx2.reshape(-1)

    zq_flat, hist = _sc_gather_hist(cb, idx)
    loss2, perp2 = _finalize(minv.reshape(minv.shape[0] * minv.shape[1], 128),
                             hist, D)

    z_q_out = jnp.transpose(zq_flat.reshape(B, H, W, D), (0, 3, 1, 2))
    return (z_q_out, loss2.reshape(()), perp2.reshape(()), idx, dist)
